# Initial kernel scaffold; baseline (speedup 1.0000x reference)
import jax
import jax.numpy as jnp
from jax.experimental import pallas as pl


def kernel(x, edge_index, batch, W1, b1, g1, be1, W2, b2, g2, be2, Wl, bl, Wo, bo):
    raise NotImplementedError("write your pallas kernel here")


# speedup vs baseline: 32.6493x; 32.6493x over previous
import functools

import jax
import jax.numpy as jnp
from jax import lax
from jax.experimental import pallas as pl
from jax.experimental.pallas import tpu as pltpu
from jax.experimental.pallas import tpu_sc as plsc

N = 100000
E = 1600000
H = 32
HH = 16
OUT = 128
G = 64
EPS = 1e-5

NC = 2
NS = 16
NW = NC * NS
NPAD = 100352
RPT = NPAD // NS
EC = 1000
EPW = E // NW
EPT = E // NS

BR = 3136
GRID = NPAD // BR


@functools.lru_cache(maxsize=None)
def _build_sc():
  mesh = plsc.VectorSubcoreMesh(
      core_axis_name="c", subcore_axis_name="s",
      num_cores=NC, num_subcores=NS)

  @functools.partial(
      pl.kernel,
      out_type=jax.ShapeDtypeStruct((NC * NPAD,), jnp.float32),
      mesh=mesh,
      scratch_types=[
          pltpu.VMEM((EC,), jnp.float32),
          pltpu.VMEM((EC,), jnp.int32),
          pltpu.VMEM_SHARED((NPAD,), jnp.float32),
      ],
  )
  def sc_deg(dst_hbm, ones_hbm, zeros_hbm, out_hbm, ones_v, idx_v, acc_sh):
    c = lax.axis_index("c")
    s = lax.axis_index("s")
    w = c * NS + s
    pltpu.sync_copy(ones_hbm, ones_v)
    pltpu.sync_copy(zeros_hbm.at[pl.ds(s * RPT, RPT)],
                    acc_sh.at[pl.ds(s * RPT, RPT)])
    plsc.subcore_barrier()

    def body(j, carry):
      base = pl.multiple_of(w * EPW + j * EC, EC)
      pltpu.sync_copy(dst_hbm.at[pl.ds(base, EC)], idx_v)
      pltpu.sync_copy(ones_v, acc_sh.at[idx_v], add=True)
      return carry

    lax.fori_loop(0, EPW // EC, body, 0)
    plsc.subcore_barrier()
    pltpu.sync_copy(acc_sh.at[pl.ds(s * RPT, RPT)],
                    out_hbm.at[pl.ds(c * NPAD + s * RPT, RPT)])

  @functools.partial(
      pl.kernel,
      out_type=jax.ShapeDtypeStruct((NC * NPAD,), jnp.float32),
      mesh=mesh,
      compiler_params=pltpu.CompilerParams(needs_layout_passes=False),
      scratch_types=[
          pltpu.VMEM((NPAD,), jnp.float32),
          pltpu.VMEM((EC,), jnp.int32),
          pltpu.VMEM((EC,), jnp.int32),
          pltpu.VMEM((EC,), jnp.float32),
          pltpu.VMEM_SHARED((NPAD,), jnp.float32),
      ],
  )
  def sc_t(src_hbm, dst_hbm, c_hbm, zeros_hbm, out_hbm,
           c_v, sidx_v, didx_v, vals_v, acc_sh):
    c = lax.axis_index("c")
    s = lax.axis_index("s")
    w = c * NS + s
    pltpu.sync_copy(c_hbm, c_v)
    pltpu.sync_copy(zeros_hbm.at[pl.ds(s * RPT, RPT)],
                    acc_sh.at[pl.ds(s * RPT, RPT)])
    plsc.subcore_barrier()

    def chunk(j, carry):
      base = pl.multiple_of(w * EPW + j * EC, EC)
      pltpu.sync_copy(src_hbm.at[pl.ds(base, EC)], sidx_v)
      pltpu.sync_copy(dst_hbm.at[pl.ds(base, EC)], didx_v)

      def gat(k, cc):
        idx = sidx_v[pl.ds(k * 16, 16)]
        vals_v[pl.ds(k * 16, 16)] = plsc.load_gather(c_v, [idx])
        return cc

      lax.fori_loop(0, EC // 16, gat, 0)
      pltpu.sync_copy(vals_v, acc_sh.at[didx_v], add=True)
      return carry

    lax.fori_loop(0, EPW // EC, chunk, 0)
    plsc.subcore_barrier()
    pltpu.sync_copy(acc_sh.at[pl.ds(s * RPT, RPT)],
                    out_hbm.at[pl.ds(c * NPAD + s * RPT, RPT)])

  @functools.partial(
      pl.kernel,
      out_type=jax.ShapeDtypeStruct((NC * NPAD, HH), jnp.float32),
      mesh=mesh,
      compiler_params=pltpu.CompilerParams(use_tc_tiling_on_sc=False),
      scratch_types=[
          pltpu.VMEM((EC,), jnp.int32),
          pltpu.VMEM((EC,), jnp.int32),
          pltpu.VMEM((EC, HH), jnp.float32),
          pltpu.VMEM_SHARED((NPAD, HH), jnp.float32),
          pltpu.SemaphoreType.DMA,
      ],
  )
  def sc_s2(src_hbm, dst_hbm, u2_hbm, zeros16_hbm, out_hbm,
            sidx_v, didx_v, rows_v, acc_sh, sem):
    c = lax.axis_index("c")
    s = lax.axis_index("s")
    off = c * NPAD
    pltpu.sync_copy(zeros16_hbm.at[pl.ds(s * RPT, RPT)],
                    acc_sh.at[pl.ds(s * RPT, RPT)])
    plsc.subcore_barrier()

    def chunk(j, carry):
      base = pl.multiple_of(s * EPT + j * EC, EC)
      pltpu.sync_copy(src_hbm.at[pl.ds(base, EC)], sidx_v)
      pltpu.sync_copy(dst_hbm.at[pl.ds(base, EC)], didx_v)

      def adj(k, cc):
        sidx_v[pl.ds(k * 16, 16)] = sidx_v[pl.ds(k * 16, 16)] + off
        return cc

      lax.fori_loop(0, EC // 16, adj, 0)
      pltpu.async_copy(u2_hbm.at[sidx_v], rows_v, sem).wait()
      pltpu.sync_copy(rows_v, acc_sh.at[didx_v], add=True)
      return carry

    lax.fori_loop(0, EPT // EC, chunk, 0)
    plsc.subcore_barrier()
    pltpu.sync_copy(acc_sh.at[pl.ds(s * RPT, RPT)],
                    out_hbm.at[pl.ds(c * NPAD + s * RPT, RPT)])

  return sc_deg, sc_t, sc_s2


def _sc_deg(dst, onesc, zeros):
  return _build_sc()[0](dst, onesc, zeros)


def _sc_t(src, dst, cvec, zeros):
  return _build_sc()[1](src, dst, cvec, zeros)


def _sc_s2(src, dst, u2, zeros16):
  return _build_sc()[2](src, dst, u2, zeros16)


def _t1_body(i0_ref, i1_ref, x_ref, dinv_ref, c_ref):
  deg = i0_ref[...] + i1_ref[...] + 1.0
  dinv = lax.rsqrt(deg)
  dinv_ref[...] = dinv
  c_ref[...] = dinv * x_ref[...]


def _t2a_body(t0_ref, t1_ref, c_ref, dinv_ref, a_ref, sa_ref, sq_ref):
  a = dinv_ref[...] * (t0_ref[...] + t1_ref[...] + c_ref[...])
  a_ref[...] = a

  @pl.when(pl.program_id(0) == 0)
  def _():
    sa_ref[...] = jnp.zeros_like(sa_ref)
    sq_ref[...] = jnp.zeros_like(sq_ref)

  sa_ref[...] += jnp.sum(a).reshape(1, 1)
  sq_ref[...] += jnp.sum(a * a).reshape(1, 1)


def _t2b_body(a_ref, dinv_ref, sa_ref, sq_ref,
              w1_ref, b1_ref, g1_ref, be1_ref, w2_ref, u2_ref):
  mean = sa_ref[0, 0] / N
  var = sq_ref[0, 0] / N - mean * mean
  w1 = w1_ref[...]
  mu1 = mean * w1 + b1_ref[...]
  inv1 = lax.rsqrt(var * (w1 * w1) + EPS)
  conv1 = a_ref[...] * w1 + b1_ref[...]
  h1 = jnp.maximum(g1_ref[...] * (conv1 - mu1) * inv1 + be1_ref[...], 0.0)
  u2 = dinv_ref[...] * jnp.dot(h1, w2_ref[...],
                               preferred_element_type=jnp.float32)
  u2_ref[0] = u2[:, :HH]
  u2_ref[1] = u2[:, HH:]


def _t3a_body(s2_ref, u2_ref, dinv_ref, b2_ref, cs_ref, cq_ref):
  x0 = s2_ref[0] + u2_ref[0]
  x1 = s2_ref[1] + u2_ref[1]
  conv2 = dinv_ref[...] * jnp.concatenate([x0, x1], axis=1) + b2_ref[...]
  rows = lax.broadcasted_iota(jnp.int32, (BR, 1), 0) + pl.program_id(0) * BR
  conv2 = jnp.where(rows < N, conv2, 0.0)

  @pl.when(pl.program_id(0) == 0)
  def _():
    cs_ref[...] = jnp.zeros_like(cs_ref)
    cq_ref[...] = jnp.zeros_like(cq_ref)

  cs_ref[...] += jnp.sum(conv2, axis=0, keepdims=True)
  cq_ref[...] += jnp.sum(conv2 * conv2, axis=0, keepdims=True)


def _t3b_body(s2_ref, u2_ref, dinv_ref, b2_ref, g2_ref, be2_ref,
              cs_ref, cq_ref, bat_ref, ps_ref, cnt_ref):
  x0 = s2_ref[0] + u2_ref[0]
  x1 = s2_ref[1] + u2_ref[1]
  conv2 = dinv_ref[...] * jnp.concatenate([x0, x1], axis=1) + b2_ref[...]
  mean = cs_ref[...] / N
  var = cq_ref[...] / N - mean * mean
  h2 = jnp.maximum(
      g2_ref[...] * (conv2 - mean) * lax.rsqrt(var + EPS) + be2_ref[...], 0.0)
  onehot = (lax.broadcasted_iota(jnp.int32, (BR, G), 1)
            == bat_ref[...]).astype(jnp.float32)

  @pl.when(pl.program_id(0) == 0)
  def _():
    ps_ref[...] = jnp.zeros_like(ps_ref)
    cnt_ref[...] = jnp.zeros_like(cnt_ref)

  dn = (((0,), (0,)), ((), ()))
  ps_ref[...] += lax.dot_general(onehot, h2, dimension_numbers=dn,
                                 preferred_element_type=jnp.float32)
  cnt_ref[...] += lax.dot_general(onehot, jnp.ones((BR, 1), jnp.float32),
                                  dimension_numbers=dn,
                                  preferred_element_type=jnp.float32)


def _t4_body(ps_ref, cnt_ref, wl_ref, bl_ref, wo_ref, bo_ref, out_ref):
  pooled = ps_ref[...] / jnp.maximum(cnt_ref[...], 1.0)
  hh = jnp.maximum(
      jnp.dot(pooled, wl_ref[...], preferred_element_type=jnp.float32)
      + bl_ref[...], 0.0)
  out_ref[...] = (jnp.dot(hh, wo_ref[...], preferred_element_type=jnp.float32)
                  + bo_ref[...])


def _col(br=BR):
  return pl.BlockSpec((br, 1), lambda i: (i, 0))


def _half():
  return pl.BlockSpec((NC, BR, HH), lambda i: (0, i, 0))


def _full(shape):
  return pl.BlockSpec(shape, lambda i: tuple(0 for _ in shape))


def kernel(x, edge_index, batch, W1, b1, g1, be1, W2, b2, g2, be2,
           Wl, bl, Wo, bo):
  src = edge_index[0]
  dst = edge_index[1]
  xp = jnp.pad(x, ((0, NPAD - N), (0, 0)))
  batp = jnp.pad(batch, (0, NPAD - N), constant_values=G).reshape(NPAD, 1)
  zeros = jnp.zeros((NPAD,), jnp.float32)
  zeros16 = jnp.zeros((NPAD, HH), jnp.float32)
  onesc = jnp.ones((EC,), jnp.float32)

  indeg = _sc_deg(dst, onesc, zeros)
  i0 = indeg[:NPAD].reshape(NPAD, 1)
  i1 = indeg[NPAD:].reshape(NPAD, 1)

  dinv, cvec = pl.pallas_call(
      _t1_body,
      grid=(GRID,),
      in_specs=[_col(), _col(), _col()],
      out_specs=[_col(), _col()],
      out_shape=[jax.ShapeDtypeStruct((NPAD, 1), jnp.float32)] * 2,
  )(i0, i1, xp)

  tpart = _sc_t(src, dst, cvec.reshape(NPAD), zeros)
  t0 = tpart[:NPAD].reshape(NPAD, 1)
  t1 = tpart[NPAD:].reshape(NPAD, 1)

  avec, sa, sq = pl.pallas_call(
      _t2a_body,
      grid=(GRID,),
      in_specs=[_col(), _col(), _col(), _col()],
      out_specs=[_col(), _full((1, 1)), _full((1, 1))],
      out_shape=[jax.ShapeDtypeStruct((NPAD, 1), jnp.float32),
                 jax.ShapeDtypeStruct((1, 1), jnp.float32),
                 jax.ShapeDtypeStruct((1, 1), jnp.float32)],
  )(t0, t1, cvec, dinv)

  u2 = pl.pallas_call(
      _t2b_body,
      grid=(GRID,),
      in_specs=[_col(), _col(), _full((1, 1)), _full((1, 1)),
                _full((1, H)), _full((1, H)), _full((1, H)), _full((1, H)),
                _full((H, H))],
      out_specs=_half(),
      out_shape=jax.ShapeDtypeStruct((NC, NPAD, HH), jnp.float32),
  )(avec, dinv, sa, sq, W1, b1.reshape(1, H), g1.reshape(1, H),
    be1.reshape(1, H), W2)

  s2 = _sc_s2(src, dst, u2.reshape(NC * NPAD, HH), zeros16)
  s2 = s2.reshape(NC, NPAD, HH)

  cs, cq = pl.pallas_call(
      _t3a_body,
      grid=(GRID,),
      in_specs=[_half(), _half(), _col(), _full((1, H))],
      out_specs=[_full((1, H)), _full((1, H))],
      out_shape=[jax.ShapeDtypeStruct((1, H), jnp.float32)] * 2,
  )(s2, u2, dinv, b2.reshape(1, H))

  ps, cnt = pl.pallas_call(
      _t3b_body,
      grid=(GRID,),
      in_specs=[_half(), _half(), _col(), _full((1, H)), _full((1, H)),
                _full((1, H)), _full((1, H)), _full((1, H)), _col()],
      out_specs=[_full((G, H)), _full((G, 1))],
      out_shape=[jax.ShapeDtypeStruct((G, H), jnp.float32),
                 jax.ShapeDtypeStruct((G, 1), jnp.float32)],
  )(s2, u2, dinv, b2.reshape(1, H), g2.reshape(1, H), be2.reshape(1, H),
    cs, cq, batp)

  out = pl.pallas_call(
      _t4_body,
      out_shape=jax.ShapeDtypeStruct((G, OUT), jnp.float32),
  )(ps, cnt, Wl, bl.reshape(1, H), Wo, bo.reshape(1, OUT))
  return out
